# baseline (speedup 1.0000x reference)
import functools

import jax
import jax.numpy as jnp
from jax import lax
from jax.experimental import pallas as pl
from jax.experimental.pallas import tpu as pltpu
from jax.experimental.pallas import tpu_sc as plsc

D = 64
MARGIN = 1.0
LANES = 16


def _fuse_body(a_ref, b_ref, o_ref):
    o_ref[:, 0:D] = a_ref[...]
    o_ref[:, D:2 * D] = b_ref[...]


@functools.cache
def _make_fuse(N: int, blk: int):
    return pl.pallas_call(
        _fuse_body,
        grid=(N // blk,),
        in_specs=[pl.BlockSpec((blk, D), lambda i: (i, 0)),
                  pl.BlockSpec((blk, D), lambda i: (i, 0))],
        out_specs=pl.BlockSpec((blk, 2 * D), lambda i: (i, 0)),
        out_shape=jax.ShapeDtypeStruct((N, 2 * D), jnp.float32),
    )


@functools.cache
def _make_score_kernel(B: int):
    NC, NS = 2, 16
    NW = NC * NS
    W = B // NW
    CH = 64
    NCHUNK = W // CH

    mesh = plsc.VectorSubcoreMesh(core_axis_name="c", subcore_axis_name="s")

    @functools.partial(
        pl.kernel,
        mesh=mesh,
        compiler_params=pltpu.CompilerParams(
            needs_layout_passes=False, use_tc_tiling_on_sc=False),
        out_type=jax.ShapeDtypeStruct((NW * LANES,), jnp.float32),
        scratch_types=[
            pltpu.VMEM((6, W), jnp.int32),
            pltpu.VMEM((6, CH, 2 * D), jnp.float32),
            pltpu.VMEM((LANES, LANES), jnp.float32),
            pltpu.VMEM((LANES,), jnp.float32),
            pltpu.SemaphoreType.DMA,
        ],
    )
    def k(ph, pr, pt, nh, nr, nt, entf, relf,
          out, idx_v, rows_v, part_v, res_v, sem):
        wid = lax.axis_index("s") * NC + lax.axis_index("c")
        base = wid * W

        for j, src in enumerate((ph, pr, pt, nh, nr, nt)):
            pltpu.sync_copy(src.at[pl.ds(base, W)], idx_v.at[j])

        plan = ((entf, 0), (relf, 1), (entf, 2),
                (entf, 3), (relf, 4), (entf, 5))

        def score(i, h_slot, r_slot, t_slot):
            p = jnp.zeros((LANES,), jnp.float32)
            for kk in range(D // LANES):
                re = pl.ds(kk * LANES, LANES)
                im = pl.ds(D + kk * LANES, LANES)
                hr = rows_v[h_slot, i, re]
                hi = rows_v[h_slot, i, im]
                rr = rows_v[r_slot, i, re]
                ri = rows_v[r_slot, i, im]
                tr = rows_v[t_slot, i, re]
                ti = rows_v[t_slot, i, im]
                p = p + rr * (hr * tr + hi * ti) + ri * (hr * ti - hi * tr)
            return p

        lane = lax.iota(jnp.int32, LANES)

        def pair_body(ii, g):
            i = g * LANES + ii
            part_v[ii, :] = score(i, 3, 4, 5) - score(i, 0, 1, 2)
            return g

        def group_body(g, acc):
            lax.fori_loop(0, LANES, pair_body, g)
            s = jnp.zeros((LANES,), jnp.float32)
            for j in range(LANES):
                s = s + plsc.load_gather(
                    part_v, [lane, jnp.full((LANES,), j, jnp.int32)])
            return acc + jnp.maximum(s + MARGIN, 0.0)

        def chunk_body(c, acc):
            copies = [
                pltpu.async_copy(
                    tbl.at[idx_v.at[jslot, pl.ds(c * CH, CH)]],
                    rows_v.at[slot], sem)
                for slot, (tbl, jslot) in enumerate(plan)
            ]
            for cp in copies:
                cp.wait()
            return lax.fori_loop(0, CH // LANES, group_body, acc)

        acc = lax.fori_loop(0, NCHUNK, chunk_body,
                            jnp.zeros((LANES,), jnp.float32))

        res_v[...] = acc
        pltpu.sync_copy(res_v, out.at[pl.ds(wid * LANES, LANES)])

    return k


def kernel(pos_exmpl, neg_exmpl, ent_real, ent_imag, rel_real, rel_imag):
    B = pos_exmpl.shape[1]
    n_ent, n_rel = ent_real.shape[0], rel_real.shape[0]
    entf = _make_fuse(n_ent, 8192)(ent_real, ent_imag)
    relf = _make_fuse(n_rel, 8192)(rel_real, rel_imag)
    k = _make_score_kernel(B)
    out = k(pos_exmpl[0], pos_exmpl[1], pos_exmpl[2],
            neg_exmpl[0], neg_exmpl[1], neg_exmpl[2],
            entf, relf)
    return jnp.sum(out)


# speedup vs baseline: 1.0484x; 1.0484x over previous
import functools

import jax
import jax.numpy as jnp
from jax import lax
from jax.experimental import pallas as pl
from jax.experimental.pallas import tpu as pltpu
from jax.experimental.pallas import tpu_sc as plsc

D = 64
MARGIN = 1.0
LANES = 16


def _fuse_body(a_ref, b_ref, o_ref):
    o_ref[:, 0:D] = a_ref[...]
    o_ref[:, D:2 * D] = b_ref[...]


@functools.cache
def _make_fuse(N: int, blk: int):
    return pl.pallas_call(
        _fuse_body,
        grid=(N // blk,),
        in_specs=[pl.BlockSpec((blk, D), lambda i: (i, 0)),
                  pl.BlockSpec((blk, D), lambda i: (i, 0))],
        out_specs=pl.BlockSpec((blk, 2 * D), lambda i: (i, 0)),
        out_shape=jax.ShapeDtypeStruct((N, 2 * D), jnp.float32),
    )


@functools.cache
def _make_score_kernel(B: int):
    NC, NS = 2, 16
    NW = NC * NS
    W = B // NW
    CH = 64
    NCHUNK = W // CH

    mesh = plsc.VectorSubcoreMesh(core_axis_name="c", subcore_axis_name="s")

    @functools.partial(
        pl.kernel,
        mesh=mesh,
        compiler_params=pltpu.CompilerParams(
            needs_layout_passes=False, use_tc_tiling_on_sc=False),
        out_type=jax.ShapeDtypeStruct((NW * LANES,), jnp.float32),
        scratch_types=[
            pltpu.VMEM((6, W), jnp.int32),
            pltpu.VMEM((8, CH, D), jnp.float32),
            pltpu.VMEM((2, CH, 2 * D), jnp.float32),
            pltpu.VMEM((LANES, LANES), jnp.float32),
            pltpu.VMEM((LANES,), jnp.float32),
            pltpu.SemaphoreType.DMA,
        ],
    )
    def k(ph, pr, pt, nh, nr, nt, ent_r, ent_i, relf,
          out, idx_v, rows_e, rows_r, part_v, res_v, sem):
        wid = lax.axis_index("s") * NC + lax.axis_index("c")
        base = wid * W

        for j, src in enumerate((ph, pr, pt, nh, nr, nt)):
            pltpu.sync_copy(src.at[pl.ds(base, W)], idx_v.at[j])

        ent_plan = ((0, ent_r, 0), (1, ent_i, 0), (2, ent_r, 2), (3, ent_i, 2),
                    (4, ent_r, 3), (5, ent_i, 3), (6, ent_r, 5), (7, ent_i, 5))
        rel_plan = ((0, 1), (1, 4))

        def score(i, hr_s, rel_s, tr_s):
            p = jnp.zeros((LANES,), jnp.float32)
            for kk in range(D // LANES):
                re = pl.ds(kk * LANES, LANES)
                im = pl.ds(D + kk * LANES, LANES)
                hr = rows_e[hr_s, i, re]
                hi = rows_e[hr_s + 1, i, re]
                rr = rows_r[rel_s, i, re]
                ri = rows_r[rel_s, i, im]
                tr = rows_e[tr_s, i, re]
                ti = rows_e[tr_s + 1, i, re]
                p = p + rr * (hr * tr + hi * ti) + ri * (hr * ti - hi * tr)
            return p

        lane = lax.iota(jnp.int32, LANES)

        def pair_body(ii, g):
            i = g * LANES + ii
            part_v[ii, :] = score(i, 4, 1, 6) - score(i, 0, 0, 2)
            return g

        def group_body(g, acc):
            lax.fori_loop(0, LANES, pair_body, g)
            s = jnp.zeros((LANES,), jnp.float32)
            for j in range(LANES):
                s = s + plsc.load_gather(
                    part_v, [lane, jnp.full((LANES,), j, jnp.int32)])
            return acc + jnp.maximum(s + MARGIN, 0.0)

        def chunk_body(c, acc):
            copies = [
                pltpu.async_copy(
                    tbl.at[idx_v.at[comp, pl.ds(c * CH, CH)]],
                    rows_e.at[slot], sem)
                for slot, tbl, comp in ent_plan
            ] + [
                pltpu.async_copy(
                    relf.at[idx_v.at[comp, pl.ds(c * CH, CH)]],
                    rows_r.at[slot], sem)
                for slot, comp in rel_plan
            ]
            for cp in copies:
                cp.wait()
            return lax.fori_loop(0, CH // LANES, group_body, acc)

        acc = lax.fori_loop(0, NCHUNK, chunk_body,
                            jnp.zeros((LANES,), jnp.float32))

        res_v[...] = acc
        pltpu.sync_copy(res_v, out.at[pl.ds(wid * LANES, LANES)])

    return k


def kernel(pos_exmpl, neg_exmpl, ent_real, ent_imag, rel_real, rel_imag):
    B = pos_exmpl.shape[1]
    n_rel = rel_real.shape[0]
    relf = _make_fuse(n_rel, 8192)(rel_real, rel_imag)
    k = _make_score_kernel(B)
    out = k(pos_exmpl[0], pos_exmpl[1], pos_exmpl[2],
            neg_exmpl[0], neg_exmpl[1], neg_exmpl[2],
            ent_real, ent_imag, relf)
    return jnp.sum(out)
